# baseline (speedup 1.0000x reference)
import functools

import jax
import jax.numpy as jnp
from jax.experimental import pallas as pl
from jax.experimental.pallas import tpu as pltpu
from jax.experimental.pallas import tpu_sc as plsc

_EPS = 1e-12
_WIN = 64


def _sc_gather_two(word_emb, ids, vorder_emb, vo):
    n = ids.shape[0]
    h = word_emb.shape[1]
    mesh = plsc.VectorSubcoreMesh(core_axis_name="c", subcore_axis_name="s")
    nw = 32
    b_per_w = n // nw
    nch = b_per_w // _WIN

    @functools.partial(
        pl.kernel,
        out_type=(
            jax.ShapeDtypeStruct((n, h), word_emb.dtype),
            jax.ShapeDtypeStruct((n, h), vorder_emb.dtype),
        ),
        mesh=mesh,
        scratch_types=[
            pltpu.VMEM((b_per_w,), jnp.int32),
            pltpu.VMEM((_WIN, h), jnp.float32),
        ],
    )
    def k(w_hbm, i_hbm, v_hbm, j_hbm, o1_hbm, o2_hbm, idx_v, rows_v):
        wid = jax.lax.axis_index("s") * 2 + jax.lax.axis_index("c")
        base = wid * b_per_w

        pltpu.sync_copy(i_hbm.at[pl.ds(base, b_per_w)], idx_v)

        @pl.loop(0, nch)
        def _(ci):
            c = ci * _WIN
            pltpu.sync_copy(w_hbm.at[idx_v.at[pl.ds(c, _WIN)]], rows_v)
            pltpu.sync_copy(rows_v, o1_hbm.at[pl.ds(base + c, _WIN)])

        pltpu.sync_copy(j_hbm.at[pl.ds(base, b_per_w)], idx_v)

        @pl.loop(0, nch)
        def _(ci):
            c = ci * _WIN
            pltpu.sync_copy(v_hbm.at[idx_v.at[pl.ds(c, _WIN)]], rows_v)
            pltpu.sync_copy(rows_v, o2_hbm.at[pl.ds(base + c, _WIN)])

    return k(word_emb, ids, vorder_emb, vo)


def _tc_body(pos_per, g1, g2, ta, pe, w0, wta, tw, tph, aw, aph, bb, tt, vseg,
             gam, bet, out):
    blk = g1.shape[0]
    x = jnp.dot(g1[...].astype(jnp.bfloat16), w0[...],
                preferred_element_type=jnp.float32)
    pe_blk = pe[pl.ds((pl.program_id(0) % pos_per) * blk, blk), :]
    td = ta[:, 0:1]
    age = ta[:, 1:2]
    vs = ta[:, 2:3]
    feat = jnp.concatenate(
        [jnp.sin(td * tw[...] + tph[...]), jnp.sin(age * aw[...] + aph[...])],
        axis=1,
    )
    x = x + jnp.dot(feat, wta[...], preferred_element_type=jnp.float32) + bb[...]
    y = jnp.tanh(x)
    vs_e = (
        jnp.where(vs == 0.0, 1.0, 0.0) * vseg[0:1, :]
        + jnp.where(vs == 1.0, 1.0, 0.0) * vseg[1:2, :]
        + jnp.where(vs == 2.0, 1.0, 0.0) * vseg[2:3, :]
    )
    emb = y + g2[...] + pe_blk + tt[...] + vs_e
    mu = jnp.mean(emb, axis=1, keepdims=True)
    d = emb - mu
    var = jnp.mean(d * d, axis=1, keepdims=True)
    out[...] = d * jax.lax.rsqrt(var + _EPS) * gam[...] + bet[...]


def _tc_forward(g1, g2, ta, pe, w0, wta, tw, tph, aw, aph, bb, tt, vseg, gam, bet,
                blk):
    n, h = g1.shape
    s = pe.shape[0]
    nblk = n // blk
    pos_per = s // blk
    grid_spec = pl.GridSpec(
        grid=(nblk,),
        in_specs=[
            pl.BlockSpec((blk, h), lambda i: (i, 0)),
            pl.BlockSpec((blk, h), lambda i: (i, 0)),
            pl.BlockSpec((blk, 4), lambda i: (i, 0)),
            pl.BlockSpec((s, h), lambda i: (0, 0)),
            pl.BlockSpec(w0.shape, lambda i: (0, 0)),
            pl.BlockSpec(wta.shape, lambda i: (0, 0)),
            pl.BlockSpec(tw.shape, lambda i: (0, 0)),
            pl.BlockSpec(tph.shape, lambda i: (0, 0)),
            pl.BlockSpec(aw.shape, lambda i: (0, 0)),
            pl.BlockSpec(aph.shape, lambda i: (0, 0)),
            pl.BlockSpec(bb.shape, lambda i: (0, 0)),
            pl.BlockSpec(tt.shape, lambda i: (0, 0)),
            pl.BlockSpec(vseg.shape, lambda i: (0, 0)),
            pl.BlockSpec(gam.shape, lambda i: (0, 0)),
            pl.BlockSpec(bet.shape, lambda i: (0, 0)),
        ],
        out_specs=pl.BlockSpec((blk, h), lambda i: (i, 0)),
    )
    return pl.pallas_call(
        functools.partial(_tc_body, pos_per),
        grid_spec=grid_spec,
        out_shape=jax.ShapeDtypeStruct((n, h), jnp.float32),
    )(g1, g2, ta, pe, w0, wta, tw, tph, aw, aph, bb, tt, vseg, gam, bet)


def kernel(input_ids, time_stamps, ages, visit_orders, visit_segments,
           word_emb, pos_emb, type_emb, vorder_emb, vseg_emb,
           time_w, time_phi, age_w, age_phi, W, b, gamma, beta):
    bsz, s = input_ids.shape
    n = bsz * s
    h = word_emb.shape[1]

    ids = input_ids.reshape(n).astype(jnp.int32)
    vo = visit_orders.reshape(n).astype(jnp.int32)
    g1, g2 = _sc_gather_two(word_emb, ids, vorder_emb, vo)

    td = jnp.concatenate(
        [time_stamps[:, 0:1] * 0, time_stamps[:, 1:] - time_stamps[:, :-1]],
        axis=-1,
    )
    ta = jnp.stack(
        [
            td.reshape(n),
            ages.reshape(n),
            visit_segments.reshape(n).astype(jnp.float32),
            jnp.zeros((n,), jnp.float32),
        ],
        axis=1,
    )

    w0 = W[:h, :].astype(jnp.bfloat16)
    wta = W[h:, :]
    out = _tc_forward(
        g1, g2, ta, pos_emb[:s], w0, wta,
        time_w, time_phi, age_w, age_phi,
        b.reshape(1, h), type_emb[0:1, :], vseg_emb,
        gamma.reshape(1, h), beta.reshape(1, h),
        blk=1024,
    )
    return out.reshape(bsz, s, h)


# speedup vs baseline: 1.2869x; 1.2869x over previous
import functools

import jax
import jax.numpy as jnp
from jax.experimental import pallas as pl
from jax.experimental.pallas import tpu as pltpu
from jax.experimental.pallas import tpu_sc as plsc

_EPS = 1e-12
_WIN = 64


def _sc_gather_two(word_emb, ids, vorder_emb, vo):
    n = ids.shape[0]
    h = word_emb.shape[1]
    mesh = plsc.VectorSubcoreMesh(core_axis_name="c", subcore_axis_name="s")
    nw = 32
    b_per_w = n // nw
    nch = b_per_w // _WIN

    @functools.partial(
        pl.kernel,
        out_type=(
            jax.ShapeDtypeStruct((n, h), word_emb.dtype),
            jax.ShapeDtypeStruct((n, h), vorder_emb.dtype),
        ),
        mesh=mesh,
        scratch_types=[
            pltpu.VMEM((b_per_w,), jnp.int32),
            pltpu.VMEM((_WIN, h), jnp.float32),
        ],
    )
    def k(w_hbm, i_hbm, v_hbm, j_hbm, o1_hbm, o2_hbm, idx_v, rows_v):
        wid = jax.lax.axis_index("s") * 2 + jax.lax.axis_index("c")
        base = wid * b_per_w

        pltpu.sync_copy(i_hbm.at[pl.ds(base, b_per_w)], idx_v)

        @pl.loop(0, nch)
        def _(ci):
            c = ci * _WIN
            pltpu.sync_copy(w_hbm.at[idx_v.at[pl.ds(c, _WIN)]], rows_v)
            pltpu.sync_copy(rows_v, o1_hbm.at[pl.ds(base + c, _WIN)])

        pltpu.sync_copy(j_hbm.at[pl.ds(base, b_per_w)], idx_v)

        @pl.loop(0, nch)
        def _(ci):
            c = ci * _WIN
            pltpu.sync_copy(v_hbm.at[idx_v.at[pl.ds(c, _WIN)]], rows_v)
            pltpu.sync_copy(rows_v, o2_hbm.at[pl.ds(base + c, _WIN)])

    return k(word_emb, ids, vorder_emb, vo)


_TWO_PI = 6.283185307179586
_INV_TWO_PI = 0.15915494309189535
_PI = 3.141592653589793


def _fast_sin(x):
    k = jnp.round(x * _INV_TWO_PI)
    r = x - k * _TWO_PI
    a = jnp.abs(r)
    m = jnp.minimum(a, _PI - a)
    m2 = m * m
    p = m * (1.0 + m2 * (-1.6666667e-1 + m2 * (8.3333333e-3
             + m2 * (-1.9841270e-4 + m2 * 2.7557319e-6))))
    return jnp.sign(r) * p


def _tc_body(pos_per, g1, g2, ta40, ta, pe, w0, wta, cw, cphi, vseg,
             gam, bet, out):
    blk = g1.shape[0]
    x = jnp.dot(g1[...].astype(jnp.bfloat16), w0[...],
                preferred_element_type=jnp.float32)
    pe_blk = pe[pl.ds((pl.program_id(0) % pos_per) * blk, blk), :]
    vs = ta[:, 2:3]
    feat = _fast_sin(ta40[...] * cw[...] + cphi[...]).astype(jnp.bfloat16)
    x = x + jnp.dot(feat, wta[...], preferred_element_type=jnp.float32)
    y = jnp.tanh(x)
    seg_iota = jax.lax.broadcasted_iota(
        jnp.int32, (1, vseg.shape[0]), 1).astype(jnp.float32)
    oh = jnp.where(vs == seg_iota, 1.0, 0.0).astype(jnp.bfloat16)
    vs_e = jnp.dot(oh, vseg[...], preferred_element_type=jnp.float32)
    emb = y + g2[...] + pe_blk + vs_e
    mu = jnp.mean(emb, axis=1, keepdims=True)
    d = emb - mu
    var = jnp.mean(d * d, axis=1, keepdims=True)
    out[...] = d * jax.lax.rsqrt(var + _EPS) * gam[...] + bet[...]


def _tc_forward(g1, g2, ta40, ta, pe, w0, wta, cw, cphi, vseg, gam, bet,
                blk):
    n, h = g1.shape
    s = pe.shape[0]
    nblk = n // blk
    pos_per = s // blk
    grid_spec = pl.GridSpec(
        grid=(nblk,),
        in_specs=[
            pl.BlockSpec((blk, h), lambda i: (i, 0)),
            pl.BlockSpec((blk, h), lambda i: (i, 0)),
            pl.BlockSpec((blk, ta40.shape[1]), lambda i: (i, 0)),
            pl.BlockSpec((blk, 4), lambda i: (i, 0)),
            pl.BlockSpec((s, h), lambda i: (0, 0)),
            pl.BlockSpec(w0.shape, lambda i: (0, 0)),
            pl.BlockSpec(wta.shape, lambda i: (0, 0)),
            pl.BlockSpec(cw.shape, lambda i: (0, 0)),
            pl.BlockSpec(cphi.shape, lambda i: (0, 0)),
            pl.BlockSpec(vseg.shape, lambda i: (0, 0)),
            pl.BlockSpec(gam.shape, lambda i: (0, 0)),
            pl.BlockSpec(bet.shape, lambda i: (0, 0)),
        ],
        out_specs=pl.BlockSpec((blk, h), lambda i: (i, 0)),
    )
    return pl.pallas_call(
        functools.partial(_tc_body, pos_per),
        grid_spec=grid_spec,
        out_shape=jax.ShapeDtypeStruct((n, h), jnp.float32),
    )(g1, g2, ta40, ta, pe, w0, wta, cw, cphi, vseg, gam, bet)


def kernel(input_ids, time_stamps, ages, visit_orders, visit_segments,
           word_emb, pos_emb, type_emb, vorder_emb, vseg_emb,
           time_w, time_phi, age_w, age_phi, W, b, gamma, beta):
    bsz, s = input_ids.shape
    n = bsz * s
    h = word_emb.shape[1]

    ids = input_ids.reshape(n).astype(jnp.int32)
    vo = visit_orders.reshape(n).astype(jnp.int32)
    g1, g2 = _sc_gather_two(word_emb, ids, vorder_emb, vo)

    td = jnp.concatenate(
        [time_stamps[:, 0:1] * 0, time_stamps[:, 1:] - time_stamps[:, :-1]],
        axis=-1,
    )
    t = time_w.shape[1]
    ta40 = jnp.concatenate(
        [
            jnp.broadcast_to(td.reshape(n, 1), (n, t)),
            jnp.broadcast_to(ages.reshape(n, 1), (n, t)),
            jnp.zeros((n, 8), jnp.float32),
        ],
        axis=1,
    )
    ta = jnp.stack(
        [
            td.reshape(n),
            ages.reshape(n),
            visit_segments.reshape(n).astype(jnp.float32),
            jnp.zeros((n,), jnp.float32),
        ],
        axis=1,
    )
    zcol = jnp.zeros((1, 8), jnp.float32)
    cw = jnp.concatenate([time_w, age_w, zcol], axis=1)
    cphi = jnp.concatenate(
        [time_phi, age_phi,
         jnp.full((1, 1), _PI / 2, jnp.float32), jnp.zeros((1, 7), jnp.float32)],
        axis=1,
    )

    w0 = W[:h, :].astype(jnp.bfloat16)
    wta = jnp.concatenate(
        [W[h:, :], b.reshape(1, h), jnp.zeros((7, h), W.dtype)], axis=0
    ).astype(jnp.bfloat16)
    vseg2 = (vseg_emb + type_emb[0:1, :]).astype(jnp.bfloat16)
    out = _tc_forward(
        g1, g2, ta40, ta, pos_emb[:s], w0, wta, cw, cphi, vseg2,
        gamma.reshape(1, h), beta.reshape(1, h),
        blk=1024,
    )
    return out.reshape(bsz, s, h)


# speedup vs baseline: 1.3614x; 1.0578x over previous
import functools

import jax
import jax.numpy as jnp
from jax.experimental import pallas as pl
from jax.experimental.pallas import tpu as pltpu
from jax.experimental.pallas import tpu_sc as plsc

_EPS = 1e-12
_WIN = 64


def _sc_gather_two(word_emb, ids, vorder_emb, vo):
    n = ids.shape[0]
    h = word_emb.shape[1]
    mesh = plsc.VectorSubcoreMesh(core_axis_name="c", subcore_axis_name="s")
    nw = 32
    b_per_w = n // nw
    nch = b_per_w // _WIN

    @functools.partial(
        pl.kernel,
        out_type=(
            jax.ShapeDtypeStruct((n, h), word_emb.dtype),
            jax.ShapeDtypeStruct((n, h), vorder_emb.dtype),
        ),
        mesh=mesh,
        scratch_types=[
            pltpu.VMEM((b_per_w,), jnp.int32),
            pltpu.VMEM((_WIN, h), jnp.float32),
        ],
    )
    def k(w_hbm, i_hbm, v_hbm, j_hbm, o1_hbm, o2_hbm, idx_v, rows_v):
        wid = jax.lax.axis_index("s") * 2 + jax.lax.axis_index("c")
        base = wid * b_per_w

        pltpu.sync_copy(i_hbm.at[pl.ds(base, b_per_w)], idx_v)

        @pl.loop(0, nch)
        def _(ci):
            c = ci * _WIN
            pltpu.sync_copy(w_hbm.at[idx_v.at[pl.ds(c, _WIN)]], rows_v)
            pltpu.sync_copy(rows_v, o1_hbm.at[pl.ds(base + c, _WIN)])

        pltpu.sync_copy(j_hbm.at[pl.ds(base, b_per_w)], idx_v)

        @pl.loop(0, nch)
        def _(ci):
            c = ci * _WIN
            pltpu.sync_copy(v_hbm.at[idx_v.at[pl.ds(c, _WIN)]], rows_v)
            pltpu.sync_copy(rows_v, o2_hbm.at[pl.ds(base + c, _WIN)])

    return k(word_emb, ids, vorder_emb, vo)


_TWO_PI = 6.283185307179586
_INV_TWO_PI = 0.15915494309189535
_PI = 3.141592653589793


def _fast_sin(x):
    k = jnp.round(x * _INV_TWO_PI)
    r = x - k * _TWO_PI
    a = jnp.abs(r)
    m = jnp.minimum(a, _PI - a)
    m2 = m * m
    p = m * (1.0 + m2 * (-1.6666667e-1 + m2 * (8.3333333e-3
             + m2 * (-1.9841270e-4 + m2 * 2.7557319e-6))))
    return jnp.sign(r) * p


def _tc_body(pos_per, has_acc, g1, g2, ta40, ta, pe, w0, wta, cw, cphi, vseg,
             gam, bet, *rest):
    out = rest[-1]
    blk = g1.shape[0]
    x = jnp.dot(g1[...].astype(jnp.bfloat16), w0[...],
                preferred_element_type=jnp.float32)
    pe_blk = pe[pl.ds((pl.program_id(0) % pos_per) * blk, blk), :]
    vs = ta[:, 2:3]
    feat = _fast_sin(ta40[...] * cw[...] + cphi[...]).astype(jnp.bfloat16)
    x = x + jnp.dot(feat, wta[...], preferred_element_type=jnp.float32)
    y = jnp.tanh(x)
    seg_iota = jax.lax.broadcasted_iota(
        jnp.int32, (1, vseg.shape[0]), 1).astype(jnp.float32)
    oh = jnp.where(vs == seg_iota, 1.0, 0.0).astype(jnp.bfloat16)
    vs_e = jnp.dot(oh, vseg[...], preferred_element_type=jnp.float32)
    emb = y + g2[...] + pe_blk + vs_e
    mu = jnp.mean(emb, axis=1, keepdims=True)
    d = emb - mu
    var = jnp.mean(d * d, axis=1, keepdims=True)
    out[...] = d * jax.lax.rsqrt(var + _EPS) * gam[...] + bet[...]


def _tc_forward_chunk(g1, g2, ta40, ta, pe, w0, wta, cw, cphi, vseg, gam, bet,
                      blk, n, blk0, acc):
    nc, h = g1.shape
    s = pe.shape[0]
    nblk = nc // blk
    pos_per = s // blk
    in_specs = [
        pl.BlockSpec((blk, h), lambda i: (i, 0)),
        pl.BlockSpec((blk, h), lambda i: (i, 0)),
        pl.BlockSpec((blk, ta40.shape[1]), lambda i: (i, 0)),
        pl.BlockSpec((blk, 4), lambda i: (i, 0)),
        pl.BlockSpec((s, h), lambda i: (0, 0)),
        pl.BlockSpec(w0.shape, lambda i: (0, 0)),
        pl.BlockSpec(wta.shape, lambda i: (0, 0)),
        pl.BlockSpec(cw.shape, lambda i: (0, 0)),
        pl.BlockSpec(cphi.shape, lambda i: (0, 0)),
        pl.BlockSpec(vseg.shape, lambda i: (0, 0)),
        pl.BlockSpec(gam.shape, lambda i: (0, 0)),
        pl.BlockSpec(bet.shape, lambda i: (0, 0)),
    ]
    args = [g1, g2, ta40, ta, pe, w0, wta, cw, cphi, vseg, gam, bet]
    io_aliases = {}
    if acc is not None:
        in_specs.append(pl.BlockSpec(memory_space=pl.ANY))
        args.append(acc)
        io_aliases = {len(args) - 1: 0}
    body = functools.partial(_tc_body, pos_per, acc is not None)
    return pl.pallas_call(
        body,
        grid=(nblk,),
        in_specs=in_specs,
        out_specs=pl.BlockSpec((blk, h), lambda i: (blk0 + i, 0)),
        out_shape=jax.ShapeDtypeStruct((n, h), jnp.float32),
        input_output_aliases=io_aliases,
    )(*args)


def kernel(input_ids, time_stamps, ages, visit_orders, visit_segments,
           word_emb, pos_emb, type_emb, vorder_emb, vseg_emb,
           time_w, time_phi, age_w, age_phi, W, b, gamma, beta):
    bsz, s = input_ids.shape
    n = bsz * s
    h = word_emb.shape[1]

    ids = input_ids.reshape(n).astype(jnp.int32)
    vo = visit_orders.reshape(n).astype(jnp.int32)

    td = jnp.concatenate(
        [time_stamps[:, 0:1] * 0, time_stamps[:, 1:] - time_stamps[:, :-1]],
        axis=-1,
    )
    t = time_w.shape[1]
    ta40 = jnp.concatenate(
        [
            jnp.broadcast_to(td.reshape(n, 1), (n, t)),
            jnp.broadcast_to(ages.reshape(n, 1), (n, t)),
            jnp.zeros((n, 8), jnp.float32),
        ],
        axis=1,
    )
    ta = jnp.stack(
        [
            td.reshape(n),
            ages.reshape(n),
            visit_segments.reshape(n).astype(jnp.float32),
            jnp.zeros((n,), jnp.float32),
        ],
        axis=1,
    )
    zcol = jnp.zeros((1, 8), jnp.float32)
    cw = jnp.concatenate([time_w, age_w, zcol], axis=1)
    cphi = jnp.concatenate(
        [time_phi, age_phi,
         jnp.full((1, 1), _PI / 2, jnp.float32), jnp.zeros((1, 7), jnp.float32)],
        axis=1,
    )

    w0 = W[:h, :].astype(jnp.bfloat16)
    wta = jnp.concatenate(
        [W[h:, :], b.reshape(1, h), jnp.zeros((7, h), W.dtype)], axis=0
    ).astype(jnp.bfloat16)
    vseg2 = (vseg_emb + type_emb[0:1, :]).astype(jnp.bfloat16)
    gam = gamma.reshape(1, h)
    bet = beta.reshape(1, h)
    pe = pos_emb[:s]

    blk = 1024
    nch = 4
    chunk = n // nch
    acc = None
    for k in range(nch):
        lo = k * chunk
        g1, g2 = _sc_gather_two(word_emb, ids[lo:lo + chunk],
                                vorder_emb, vo[lo:lo + chunk])
        acc = _tc_forward_chunk(
            g1, g2, ta40[lo:lo + chunk], ta[lo:lo + chunk], pe,
            w0, wta, cw, cphi, vseg2, gam, bet,
            blk=blk, n=n, blk0=lo // blk, acc=acc,
        )
    return acc.reshape(bsz, s, h)
